# Initial kernel scaffold; baseline (speedup 1.0000x reference)
import jax
import jax.numpy as jnp
from jax.experimental import pallas as pl


def kernel(data_A, data_BP, nnf_sr, nnf_rs, curr_layer):
    raise NotImplementedError("write your pallas kernel here")


# speedup vs baseline: 1.5892x; 1.5892x over previous
import functools

import jax
import jax.numpy as jnp
from jax import lax
from jax.experimental import pallas as pl
from jax.experimental.pallas import tpu as pltpu
from jax.experimental.pallas import tpu_sc as plsc

H = 128
W = 128
P = H * W
C = 256
CK = 64
NCK = C // CK
PAD = P
WS = 1.0 / P
WR = 2.0 / P
NW = 32
PPT = P // NW
NB = 72
OFFS = [(dy, dx) for dy in (-1, 0, 1) for dx in (-1, 0, 1)]

_mesh = plsc.VectorSubcoreMesh(core_axis_name="c", subcore_axis_name="s")


@functools.partial(
    pl.kernel,
    mesh=_mesh,
    compiler_params=pltpu.CompilerParams(use_tc_tiling_on_sc=False),
    out_type=[
        jax.ShapeDtypeStruct((NCK, 2, P, CK), jnp.float32),
        jax.ShapeDtypeStruct((2, P, 16), jnp.float32),
    ],
    scratch_types=[
        pltpu.VMEM_SHARED((P, CK), jnp.float32),
        pltpu.VMEM_SHARED((P, 16), jnp.float32),
        pltpu.VMEM((PPT,), jnp.int32),
        pltpu.VMEM((PPT,), jnp.int32),
        pltpu.VMEM((PPT,), jnp.int32),
        pltpu.VMEM((PPT,), jnp.int32),
        pltpu.VMEM((NB * 128,), jnp.int32),
        pltpu.VMEM((NB, 128), jnp.int32),
        pltpu.VMEM((128,), jnp.int32),
        pltpu.VMEM((128, CK), jnp.float32),
        pltpu.VMEM((128, CK), jnp.float32),
        pltpu.VMEM((128, 16), jnp.float32),
        pltpu.VMEM((128, 16), jnp.float32),
    ],
)
def _sc_vote(ref8, n1y, n1x, n2y, n2x, zsrc, wtab, acc_out, w_out,
             guide_sp, w_sp, n1y_v, n1x_v, n2y_v, n2x_v, gbuf, tbuf, wibuf,
             rows_v, zrows_v, wrows_v, zw_v):
    cid = lax.axis_index("c")
    sid = lax.axis_index("s")
    wid = sid * 2 + cid
    base = wid * PPT

    pltpu.sync_copy(n1y.at[pl.ds(base, PPT)], n1y_v)
    pltpu.sync_copy(n1x.at[pl.ds(base, PPT)], n1x_v)
    pltpu.sync_copy(n2y.at[pl.ds(base, PPT)], n2y_v)
    pltpu.sync_copy(n2x.at[pl.ds(base, PPT)], n2x_v)
    pltpu.sync_copy(zsrc, zrows_v)

    zv16 = jnp.zeros((16,), jnp.float32)

    def zero_zw(i, carry):
        zw_v[i, pl.ds(0, 16)] = zv16
        return carry

    lax.fori_loop(0, 128, zero_zw, 0)

    iot = lax.iota(jnp.int32, 16)

    for d in range(2):
        ny, nx = (n1y_v, n1x_v) if d == 0 else (n2y_v, n2x_v)
        wv = WS if d == 0 else WR
        for oi, (dy, dx) in enumerate(OFFS):
            q = d * 9 + oi

            def build(j, carry, d=d, dy=dy, dx=dx, q=q, ny=ny, nx=nx, wv=wv):
                p = base + j * 16 + iot
                py = lax.shift_right_logical(p, 7)
                px = lax.bitwise_and(p, W - 1)
                my = ny[pl.ds(j * 16, 16)]
                mx = nx[pl.ds(j * 16, 16)]
                if d == 0:
                    ty = py + dy
                    tx = px + dx
                    gy = my + dy
                    gx = mx + dx
                else:
                    ty = my + dy
                    tx = mx + dx
                    gy = py + dy
                    gx = px + dx
                m = ((ty >= 0) & (ty < H) & (tx >= 0) & (tx < W)
                     & (gy >= 0) & (gy < H) & (gx >= 0) & (gx < W))
                t = jnp.where(m, ty * W + tx, 0)
                g = jnp.where(m, gy * W + gx, PAD)
                gbuf[pl.ds(q * PPT + j * 16, 16)] = g
                b = q * 4 + lax.shift_right_logical(j, 3)
                col = lax.bitwise_and(j, 7) * 16
                tbuf[b, pl.ds(col, 16)] = t
                return carry

            lax.fori_loop(0, PPT // 16, build, 0)

    def zero_slice(k, carry):
        pltpu.sync_copy(zrows_v, guide_sp.at[pl.ds(sid * 1024 + k * 128, 128)])
        return carry

    def zero_wslice(k, carry):
        pltpu.sync_copy(zw_v, w_sp.at[pl.ds(sid * 1024 + k * 128, 128)])
        return carry

    lax.fori_loop(0, 8, zero_slice, 0)
    lax.fori_loop(0, 8, zero_wslice, 0)

    for ck in range(NCK):
        plsc.subcore_barrier()
        if ck == 0:
            def wvote(b, carry, dbase=0):
                def mk(jj, c2):
                    g16 = gbuf[pl.ds(b * 128 + jj * 16, 16)]
                    wibuf[pl.ds(jj * 16, 16)] = jnp.where(
                        g16 == PAD, dbase + 1, dbase)
                    return c2

                lax.fori_loop(0, 8, mk, 0)
                pltpu.sync_copy(wtab.at[wibuf], wrows_v)
                pltpu.sync_copy(wrows_v, w_sp.at[tbuf.at[b]], add=True)
                return carry

            lax.fori_loop(0, NB // 2, wvote, 0)
            lax.fori_loop(NB // 2, NB,
                          functools.partial(wvote, dbase=2), 0)

        def vote(b, carry, ck=ck):
            pltpu.sync_copy(ref8.at[ck].at[gbuf.at[pl.ds(b * 128, 128)]], rows_v)
            pltpu.sync_copy(rows_v, guide_sp.at[tbuf.at[b]], add=True)
            return carry

        def vote2(b, carry, ck=ck):
            pltpu.sync_copy(
                ref8.at[NCK + ck].at[gbuf.at[pl.ds(b * 128, 128)]], rows_v)
            pltpu.sync_copy(rows_v, guide_sp.at[tbuf.at[b]], add=True)
            return carry

        lax.fori_loop(0, NB // 2, vote, 0)
        lax.fori_loop(NB // 2, NB, vote2, 0)
        plsc.subcore_barrier()

        def dump(k, carry, ck=ck):
            off = sid * 1024 + k * 128
            pltpu.sync_copy(guide_sp.at[pl.ds(off, 128)], rows_v)
            pltpu.sync_copy(rows_v, acc_out.at[ck, cid, pl.ds(off, 128)])
            return carry

        lax.fori_loop(0, 8, dump, 0)
        if ck == 0:
            def wdump(k, carry):
                off = sid * 1024 + k * 128
                pltpu.sync_copy(w_sp.at[pl.ds(off, 128)], wrows_v)
                pltpu.sync_copy(wrows_v, w_out.at[cid, pl.ds(off, 128)])
                return carry

            lax.fori_loop(0, 8, wdump, 0)
        if ck < NCK - 1:
            lax.fori_loop(0, 8, zero_slice, 0)


def _merge_body(acc_ref, w_ref, out_ref):
    w = w_ref[0, :, 0] + w_ref[1, :, 0]
    w = jnp.where(w == 0.0, 1.0, w)
    inv = (1.0 / w)[:, None]
    for ck in range(NCK):
        g = acc_ref[ck, 0] + acc_ref[ck, 1]
        out_ref[:, ck * CK:(ck + 1) * CK] = g * inv


_merge = pl.pallas_call(
    _merge_body,
    grid=(16,),
    in_specs=[
        pl.BlockSpec((NCK, 2, 1024, CK), lambda i: (0, 0, i, 0)),
        pl.BlockSpec((2, 1024, 16), lambda i: (0, i, 0)),
    ],
    out_specs=pl.BlockSpec((1024, C), lambda i: (i, 0)),
    out_shape=jax.ShapeDtypeStruct((P, C), jnp.float32),
)


def kernel(data_A, data_BP, nnf_sr, nnf_rs, curr_layer):
    refT = data_BP[0].reshape(C, P).T
    ref_pad = jnp.concatenate(
        [refT, jnp.zeros((1, C), jnp.float32)], axis=0)
    ref4 = ref_pad.reshape(P + 1, NCK, CK).transpose(1, 0, 2)
    ref8 = jnp.concatenate([WS * ref4, WR * ref4], axis=0)
    n1y = nnf_sr[..., 0].reshape(P).astype(jnp.int32)
    n1x = nnf_sr[..., 1].reshape(P).astype(jnp.int32)
    n2y = nnf_rs[..., 0].reshape(P).astype(jnp.int32)
    n2x = nnf_rs[..., 1].reshape(P).astype(jnp.int32)
    zsrc = jnp.zeros((128, CK), jnp.float32)
    wtab = jnp.zeros((4, 16), jnp.float32)
    wtab = wtab.at[0].set(WS).at[2].set(WR)

    acc, wparts = _sc_vote(ref8, n1y, n1x, n2y, n2x, zsrc, wtab)
    guide_flat = _merge(acc, wparts)
    guide = guide_flat.T.reshape(C, H, W)
    return guide, data_A, data_BP
